# Initial kernel scaffold; baseline (speedup 1.0000x reference)
import jax
import jax.numpy as jnp
from jax import lax
from jax.experimental import pallas as pl
from jax.experimental.pallas import tpu as pltpu
from jax.experimental.pallas import tpu_sc as plsc

N = 10000
D = 128
H = 16
C = 2
E = 320000

NC = 2
NS = 16
NW = NC * NS
CH = 128

PAD_N = 10240
DUMMY = N
ROWS_PT = PAD_N // NS
E_PAD = -(-E // (NW * CH)) * (NW * CH)
EPW = E_PAD // NW


def _sc_mesh():
    return plsc.VectorSubcoreMesh(
        core_axis_name="c", subcore_axis_name="s", num_cores=NC, num_subcores=NS
    )


def _deg_body(dst_hbm, z_hbm, out_hbm, ones_v, didx_v, acc_sh):
    c = lax.axis_index("c")
    s = lax.axis_index("s")
    wid = c * NS + s
    for i in range(CH // 16):
        ones_v[pl.ds(i * 16, 16)] = jnp.ones((16,), jnp.float32)
    pltpu.sync_copy(z_hbm.at[pl.ds(s * ROWS_PT, ROWS_PT)],
                    acc_sh.at[pl.ds(s * ROWS_PT, ROWS_PT)])
    plsc.subcore_barrier()

    def step(j, carry):
        base = pl.multiple_of(wid * EPW + j * CH, CH)
        pltpu.sync_copy(dst_hbm.at[pl.ds(base, CH)], didx_v)
        pltpu.sync_copy(ones_v, acc_sh.at[didx_v], add=True)
        return carry

    lax.fori_loop(0, EPW // CH, step, 0)
    plsc.subcore_barrier()
    pltpu.sync_copy(acc_sh.at[pl.ds(s * ROWS_PT, ROWS_PT)],
                    out_hbm.at[c, pl.ds(s * ROWS_PT, ROWS_PT)])


_deg_call = pl.kernel(
    _deg_body,
    out_type=jax.ShapeDtypeStruct((NC, PAD_N), jnp.float32),
    mesh=_sc_mesh(),
    scratch_types=[
        pltpu.VMEM((CH,), jnp.float32),
        pltpu.VMEM((CH,), jnp.int32),
        pltpu.VMEM_SHARED((PAD_N,), jnp.float32),
    ],
)


def _make_scatter(width):
    def body(src_hbm, dst_hbm, g_hbm, z_hbm, out_hbm, sidx_v, didx_v, rows_v,
             acc_sh, sem):
        c = lax.axis_index("c")
        s = lax.axis_index("s")
        wid = c * NS + s
        pltpu.sync_copy(z_hbm.at[pl.ds(s * ROWS_PT, ROWS_PT)],
                        acc_sh.at[pl.ds(s * ROWS_PT, ROWS_PT)])
        plsc.subcore_barrier()

        def step(j, carry):
            base = pl.multiple_of(wid * EPW + j * CH, CH)
            pltpu.sync_copy(src_hbm.at[pl.ds(base, CH)], sidx_v)
            pltpu.sync_copy(dst_hbm.at[pl.ds(base, CH)], didx_v)
            pltpu.async_copy(g_hbm.at[sidx_v], rows_v, sem).wait()
            pltpu.sync_copy(rows_v, acc_sh.at[didx_v], add=True)
            return carry

        lax.fori_loop(0, EPW // CH, step, 0)
        plsc.subcore_barrier()
        pltpu.sync_copy(acc_sh.at[pl.ds(s * ROWS_PT, ROWS_PT)],
                        out_hbm.at[c, pl.ds(s * ROWS_PT, ROWS_PT)])

    return pl.kernel(
        body,
        out_type=jax.ShapeDtypeStruct((NC, PAD_N, width), jnp.float32),
        mesh=_sc_mesh(),
        scratch_types=[
            pltpu.VMEM((CH,), jnp.int32),
            pltpu.VMEM((CH,), jnp.int32),
            pltpu.VMEM((CH, width), jnp.float32),
            pltpu.VMEM_SHARED((PAD_N, width), jnp.float32),
            pltpu.SemaphoreType.DMA,
        ],
    )


_scatter_h = _make_scatter(H)
_scatter_c = _make_scatter(C)


def _dense1_body(dp_ref, x_ref, w1_ref, g1_ref, dis_ref):
    dp = dp_ref[...]
    deg = dp[:, 0:1] + dp[:, 1:2] + 1.0
    dis = lax.rsqrt(deg)
    p1 = jnp.dot(x_ref[...], w1_ref[...], preferred_element_type=jnp.float32)
    g1_ref[...] = dis * p1
    dis_ref[...] = dis


def _dense1(dp, xp, w1):
    return pl.pallas_call(
        _dense1_body,
        out_shape=[
            jax.ShapeDtypeStruct((PAD_N, H), jnp.float32),
            jax.ShapeDtypeStruct((PAD_N, 1), jnp.float32),
        ],
    )(dp, xp, w1)


def _dense2_body(s_ref, g1_ref, dis_ref, b1_ref, w2_ref, g2_ref):
    ssum = s_ref[0] + s_ref[1] + g1_ref[...]
    h1 = dis_ref[...] * ssum + b1_ref[...]
    hr = jnp.maximum(h1, 0.0)
    g2_ref[...] = dis_ref[...] * jnp.dot(
        hr, w2_ref[...], preferred_element_type=jnp.float32)


def _dense2(s1, g1, dis, b1, w2):
    return pl.pallas_call(
        _dense2_body,
        out_shape=jax.ShapeDtypeStruct((PAD_N, C), jnp.float32),
    )(s1, g1, dis, b1, w2)


def _dense3_body(s_ref, g2_ref, dis_ref, b2_ref, o_ref):
    ssum = s_ref[0] + s_ref[1] + g2_ref[...]
    o_ref[...] = dis_ref[...] * ssum + b2_ref[...]


def _dense3(s2, g2, dis, b2):
    return pl.pallas_call(
        _dense3_body,
        out_shape=jax.ShapeDtypeStruct((PAD_N, C), jnp.float32),
    )(s2, g2, dis, b2)


def kernel(x, edge_index, W1, b1, W2, b2):
    ei = edge_index.astype(jnp.int32)
    fill = jnp.full((E_PAD - E,), DUMMY, jnp.int32)
    src = jnp.concatenate([ei[0], fill])
    dst = jnp.concatenate([ei[1], fill])
    xp = jnp.pad(x, ((0, PAD_N - N), (0, 0)))

    z1 = jnp.zeros((PAD_N,), jnp.float32)
    zh = jnp.zeros((PAD_N, H), jnp.float32)
    zc = jnp.zeros((PAD_N, C), jnp.float32)

    deg2 = _deg_call(dst, z1)
    g1, dis = _dense1(deg2.T, xp, W1)
    s1 = _scatter_h(src, dst, g1, zh)
    g2 = _dense2(s1, g1, dis, b1.reshape(1, H), W2)
    s2 = _scatter_c(src, dst, g2, zc)
    out = _dense3(s2, g2, dis, b2.reshape(1, C))
    return out[:N]


# speedup vs baseline: 21.4762x; 21.4762x over previous
import jax
import jax.numpy as jnp
from jax import lax
from jax.experimental import pallas as pl
from jax.experimental.pallas import tpu as pltpu
from jax.experimental.pallas import tpu_sc as plsc

N = 10000
D = 128
H = 16
C = 2
E = 320000

NC = 2
NS = 16
NW = NC * NS
CH = 128

PAD_N = 10240
DUMMY = N
ROWS_PT = PAD_N // NS
E_PAD = -(-E // (NW * CH)) * (NW * CH)
EPW = E_PAD // NW


def _sc_mesh():
    return plsc.VectorSubcoreMesh(
        core_axis_name="c", subcore_axis_name="s", num_cores=NC, num_subcores=NS
    )


def _deg_body(dst_hbm, z_hbm, out_hbm, ones_v, didx_v, acc_sh):
    c = lax.axis_index("c")
    s = lax.axis_index("s")
    wid = c * NS + s
    for i in range(CH // 16):
        ones_v[pl.ds(i * 16, 16)] = jnp.ones((16,), jnp.float32)
    pltpu.sync_copy(z_hbm.at[pl.ds(s * ROWS_PT, ROWS_PT)],
                    acc_sh.at[pl.ds(s * ROWS_PT, ROWS_PT)])
    plsc.subcore_barrier()

    def step(j, carry):
        base = pl.multiple_of(wid * EPW + j * CH, CH)
        pltpu.sync_copy(dst_hbm.at[pl.ds(base, CH)], didx_v)
        pltpu.sync_copy(ones_v, acc_sh.at[didx_v], add=True)
        return carry

    lax.fori_loop(0, EPW // CH, step, 0)
    plsc.subcore_barrier()
    pltpu.sync_copy(acc_sh.at[pl.ds(s * ROWS_PT, ROWS_PT)],
                    out_hbm.at[c, pl.ds(s * ROWS_PT, ROWS_PT)])


_deg_call = pl.kernel(
    _deg_body,
    out_type=jax.ShapeDtypeStruct((NC, PAD_N), jnp.float32),
    mesh=_sc_mesh(),
    compiler_params=pltpu.CompilerParams(use_tc_tiling_on_sc=False),
    scratch_types=[
        pltpu.VMEM((CH,), jnp.float32),
        pltpu.VMEM((CH,), jnp.int32),
        pltpu.VMEM_SHARED((PAD_N,), jnp.float32),
    ],
)


def _make_scatter(width):
    def body(src_hbm, dst_hbm, g_hbm, z_hbm, out_hbm, sidx_v, didx_v, rows_v,
             acc_sh, sem):
        c = lax.axis_index("c")
        s = lax.axis_index("s")
        wid = c * NS + s
        pltpu.sync_copy(z_hbm.at[pl.ds(s * ROWS_PT, ROWS_PT)],
                        acc_sh.at[pl.ds(s * ROWS_PT, ROWS_PT)])
        plsc.subcore_barrier()

        def step(j, carry):
            base = pl.multiple_of(wid * EPW + j * CH, CH)
            pltpu.sync_copy(src_hbm.at[pl.ds(base, CH)], sidx_v)
            pltpu.sync_copy(dst_hbm.at[pl.ds(base, CH)], didx_v)
            pltpu.async_copy(g_hbm.at[sidx_v], rows_v, sem).wait()
            pltpu.sync_copy(rows_v, acc_sh.at[didx_v], add=True)
            return carry

        lax.fori_loop(0, EPW // CH, step, 0)
        plsc.subcore_barrier()
        pltpu.sync_copy(acc_sh.at[pl.ds(s * ROWS_PT, ROWS_PT)],
                        out_hbm.at[c, pl.ds(s * ROWS_PT, ROWS_PT)])

    return pl.kernel(
        body,
        out_type=jax.ShapeDtypeStruct((NC, PAD_N, width), jnp.float32),
        mesh=_sc_mesh(),
        compiler_params=pltpu.CompilerParams(use_tc_tiling_on_sc=False),
        scratch_types=[
            pltpu.VMEM((CH,), jnp.int32),
            pltpu.VMEM((CH,), jnp.int32),
            pltpu.VMEM((CH, width), jnp.float32),
            pltpu.VMEM_SHARED((PAD_N, width), jnp.float32),
            pltpu.SemaphoreType.DMA,
        ],
    )


_scatter_h = _make_scatter(H)


def _dense1_body(dp_ref, x_ref, w1_ref, g1_ref, dis_ref):
    dp = dp_ref[...]
    deg = dp[:, 0:1] + dp[:, 1:2] + 1.0
    dis = lax.rsqrt(deg)
    p1 = jnp.dot(x_ref[...], w1_ref[...], preferred_element_type=jnp.float32)
    g1_ref[...] = dis * p1
    dis_ref[...] = dis


def _dense1(dp, xp, w1):
    return pl.pallas_call(
        _dense1_body,
        out_shape=[
            jax.ShapeDtypeStruct((PAD_N, H), jnp.float32),
            jax.ShapeDtypeStruct((PAD_N, 1), jnp.float32),
        ],
    )(dp, xp, w1)


def _dense2_body(s_ref, g1_ref, dis_ref, b1_ref, g2_ref):
    ssum = s_ref[0] + s_ref[1] + g1_ref[...]
    h1 = dis_ref[...] * ssum + b1_ref[...]
    hr = jnp.maximum(h1, 0.0)
    g2_ref[...] = dis_ref[...] * hr


def _dense2(s1, g1, dis, b1):
    return pl.pallas_call(
        _dense2_body,
        out_shape=jax.ShapeDtypeStruct((PAD_N, H), jnp.float32),
    )(s1, g1, dis, b1)


def _dense3_body(s_ref, g2_ref, dis_ref, w2_ref, b2_ref, o_ref):
    ssum = s_ref[0] + s_ref[1] + g2_ref[...]
    ah = dis_ref[...] * ssum
    o_ref[...] = jnp.dot(ah, w2_ref[...],
                         preferred_element_type=jnp.float32) + b2_ref[...]


def _dense3(s2, g2, dis, w2, b2):
    return pl.pallas_call(
        _dense3_body,
        out_shape=jax.ShapeDtypeStruct((PAD_N, C), jnp.float32),
    )(s2, g2, dis, w2, b2)


def kernel(x, edge_index, W1, b1, W2, b2):
    ei = edge_index.astype(jnp.int32)
    fill = jnp.full((E_PAD - E,), DUMMY, jnp.int32)
    src = jnp.concatenate([ei[0], fill])
    dst = jnp.concatenate([ei[1], fill])
    xp = jnp.pad(x, ((0, PAD_N - N), (0, 0)))

    z1 = jnp.zeros((PAD_N,), jnp.float32)
    zh = jnp.zeros((PAD_N, H), jnp.float32)

    deg2 = _deg_call(dst, z1)
    g1, dis = _dense1(deg2.T, xp, W1)
    s1 = _scatter_h(src, dst, g1, zh)
    g2 = _dense2(s1, g1, dis, b1.reshape(1, H))
    s2 = _scatter_h(src, dst, g2, zh)
    out = _dense3(s2, g2, dis, W2, b2.reshape(1, C))
    return out[:N]


# speedup vs baseline: 32.8882x; 1.5314x over previous
import jax
import jax.numpy as jnp
from jax import lax
from jax.experimental import pallas as pl
from jax.experimental.pallas import tpu as pltpu
from jax.experimental.pallas import tpu_sc as plsc

N = 10000
D = 128
H = 16
C = 2
E = 320000

NC = 2
NS = 16
NW = NC * NS
CH = 128

PAD_N = 10240
DUMMY = N
ROWS_PT = PAD_N // NS
_NCH_RAW = -(-E // (NW * CH))
NCH = ((_NCH_RAW + 7) // 8) * 8
E_PAD = NW * CH * NCH
EPW = E_PAD // NW


def _sc_mesh():
    return plsc.VectorSubcoreMesh(
        core_axis_name="c", subcore_axis_name="s", num_cores=NC, num_subcores=NS
    )


NB = 4
RG = 8
LEAD = 4


def _deg_body(dst_hbm, out_hbm, didx_v, deg_v):
    c = lax.axis_index("c")
    s = lax.axis_index("s")
    wid = c * NS + s
    pltpu.sync_copy(dst_hbm.at[wid], didx_v)

    def zbody(i, carry):
        for u in range(8):
            deg_v[pl.ds((i * 8 + u) * 16, 16)] = jnp.zeros((16,), jnp.float32)
        return carry

    lax.fori_loop(0, PAD_N // 128, zbody, 0)
    ones = jnp.ones((16,), jnp.float32)

    def abody(i, carry):
        for u in range(8):
            idx = didx_v[pl.ds((i * 8 + u) * 16, 16)]
            plsc.addupdate_scatter(deg_v, [idx], ones)
        return carry

    lax.fori_loop(0, EPW // 128, abody, 0)
    pltpu.sync_copy(deg_v, out_hbm.at[c, s])


_deg_call = pl.kernel(
    _deg_body,
    out_type=jax.ShapeDtypeStruct((NC, NS, PAD_N), jnp.float32),
    mesh=_sc_mesh(),
    compiler_params=pltpu.CompilerParams(use_tc_tiling_on_sc=False,
                                         needs_layout_passes=False),
    scratch_types=[
        pltpu.VMEM((EPW,), jnp.int32),
        pltpu.VMEM((PAD_N,), jnp.float32),
    ],
)


def _make_scatter(width):
    def body(src_hbm, dst_hbm, g_hbm, z_hbm, out_hbm, rows_v,
             e0, e1, e2, e3, e4, e5, e6, e7,
             d0, d1, d2, d3, d4, d5, d6, d7,
             acc_sh, gsem, ssem, isem, jsem):
        sidx = (e0, e1, e2, e3, e4, e5, e6, e7)
        didx = (d0, d1, d2, d3, d4, d5, d6, d7)
        c = lax.axis_index("c")
        s = lax.axis_index("s")
        wid = c * NS + s
        for k in range(NB):
            pltpu.sync_copy(
                z_hbm.at[pl.ds(s * ROWS_PT, ROWS_PT)],
                acc_sh.at[pl.ds(k * PAD_N + s * ROWS_PT, ROWS_PT)])
        plsc.subcore_barrier()

        for b in range(RG):
            pltpu.async_copy(src_hbm.at[wid, pl.ds(b * CH, CH)], sidx[b], jsem.at[b])
        for b in range(LEAD):
            pltpu.make_async_copy(src_hbm.at[wid, pl.ds(b * CH, CH)],
                                  sidx[b], jsem.at[b]).wait()
            pltpu.async_copy(g_hbm.at[sidx[b]], rows_v.at[b], gsem.at[b])
            pltpu.async_copy(dst_hbm.at[wid, pl.ds(b * CH, CH)], didx[b], isem.at[b])

        def outer(t, carry):
            for u in range(RG):
                j = t * RG + u
                k = u % NB
                uf = (u + LEAD) % RG
                pltpu.make_async_copy(g_hbm.at[sidx[u]], rows_v.at[u],
                                      gsem.at[u]).wait()
                pltpu.make_async_copy(dst_hbm.at[wid, pl.ds(j * CH, CH)],
                                      didx[u], isem.at[u]).wait()
                if u < NB:
                    @pl.when(t < NCH // RG - 1)
                    def _():
                        pltpu.async_copy(
                            src_hbm.at[wid, pl.ds((j + RG) * CH, CH)],
                            sidx[u], jsem.at[u])
                else:
                    @pl.when(t < NCH // RG - 1)
                    def _():
                        pltpu.async_copy(
                            src_hbm.at[wid, pl.ds((j + RG) * CH, CH)],
                            sidx[u], jsem.at[u])
                if u < NB:
                    @pl.when(t >= 1)
                    def _():
                        pltpu.make_async_copy(
                            rows_v.at[uf], acc_sh.at[didx[uf]],
                            ssem.at[k]).wait()
                else:
                    pltpu.make_async_copy(
                        rows_v.at[uf], acc_sh.at[didx[uf]],
                        ssem.at[k]).wait()
                pltpu.async_copy(rows_v.at[u], acc_sh.at[didx[u]],
                                 ssem.at[k], add=True)
                f = j + LEAD
                if u < NB:
                    pltpu.make_async_copy(src_hbm.at[wid, pl.ds(f * CH, CH)],
                                          sidx[uf], jsem.at[uf]).wait()
                    pltpu.async_copy(g_hbm.at[sidx[uf]], rows_v.at[uf],
                                     gsem.at[uf])
                    pltpu.async_copy(dst_hbm.at[wid, pl.ds(f * CH, CH)],
                                     didx[uf], isem.at[uf])
                else:
                    @pl.when(t < NCH // RG - 1)
                    def _():
                        pltpu.make_async_copy(
                            src_hbm.at[wid, pl.ds(f * CH, CH)],
                            sidx[uf], jsem.at[uf]).wait()
                        pltpu.async_copy(g_hbm.at[sidx[uf]], rows_v.at[uf],
                                         gsem.at[uf])
                        pltpu.async_copy(dst_hbm.at[wid, pl.ds(f * CH, CH)],
                                         didx[uf], isem.at[uf])
            return carry

        lax.fori_loop(0, NCH // RG, outer, 0)
        for b in range(NB):
            j = NCH - NB + b
            pltpu.make_async_copy(rows_v.at[j % RG],
                                  acc_sh.at[didx[j % RG]],
                                  ssem.at[j % NB]).wait()
        plsc.subcore_barrier()
        for k in range(NB):
            pltpu.sync_copy(
                acc_sh.at[pl.ds(k * PAD_N + s * ROWS_PT, ROWS_PT)],
                out_hbm.at[c, k, pl.ds(s * ROWS_PT, ROWS_PT)])

    return pl.kernel(
        body,
        out_type=jax.ShapeDtypeStruct((NC, NB, PAD_N, width), jnp.float32),
        mesh=_sc_mesh(),
        compiler_params=pltpu.CompilerParams(use_tc_tiling_on_sc=False),
        scratch_types=[
            pltpu.VMEM((RG, CH, width), jnp.float32)]
            + [pltpu.VMEM((CH,), jnp.int32) for _ in range(2 * RG)]
            + [pltpu.VMEM_SHARED((NB * PAD_N, width), jnp.float32)]
            + [
            pltpu.SemaphoreType.DMA((RG,)),
            pltpu.SemaphoreType.DMA((NB,)),
            pltpu.SemaphoreType.DMA((RG,)),
            pltpu.SemaphoreType.DMA((RG,)),
        ],
    )


_scatter_h = _make_scatter(H)


def _dense1_body(dp_ref, x_ref, w1_ref, g1_ref, dis_ref):
    dp = dp_ref[...]
    ones_w = jnp.ones((NW, 1), jnp.float32)
    deg = lax.dot_general(dp, ones_w, (((0,), (0,)), ((), ())),
                          preferred_element_type=jnp.float32) + 1.0
    dis = lax.rsqrt(deg)
    p1 = jnp.dot(x_ref[...], w1_ref[...], preferred_element_type=jnp.float32)
    g1_ref[...] = dis * p1
    dis_ref[...] = dis


def _dense1(dp, xp, w1):
    return pl.pallas_call(
        _dense1_body,
        out_shape=[
            jax.ShapeDtypeStruct((PAD_N, H), jnp.float32),
            jax.ShapeDtypeStruct((PAD_N, 1), jnp.float32),
        ],
    )(dp, xp, w1)


def _dense2_body(s_ref, g1_ref, dis_ref, b1_ref, g2_ref):
    ssum = g1_ref[...]
    for i in range(NC * NB):
        ssum = ssum + s_ref[i]
    h1 = dis_ref[...] * ssum + b1_ref[...]
    hr = jnp.maximum(h1, 0.0)
    g2_ref[...] = dis_ref[...] * hr


def _dense2(s1, g1, dis, b1):
    return pl.pallas_call(
        _dense2_body,
        out_shape=jax.ShapeDtypeStruct((PAD_N, H), jnp.float32),
    )(s1, g1, dis, b1)


def _dense3_body(s_ref, g2_ref, dis_ref, w2_ref, b2_ref, o_ref):
    ssum = g2_ref[...]
    for i in range(NC * NB):
        ssum = ssum + s_ref[i]
    ah = dis_ref[...] * ssum
    o_ref[...] = jnp.dot(ah, w2_ref[...],
                         preferred_element_type=jnp.float32) + b2_ref[...]


def _dense3(s2, g2, dis, w2, b2):
    return pl.pallas_call(
        _dense3_body,
        out_shape=jax.ShapeDtypeStruct((PAD_N, C), jnp.float32),
    )(s2, g2, dis, w2, b2)


def kernel(x, edge_index, W1, b1, W2, b2):
    ei = edge_index.astype(jnp.int32)
    fill = jnp.full((E_PAD - E,), DUMMY, jnp.int32)
    src = jnp.concatenate([ei[0], fill])
    dst = jnp.concatenate([ei[1], fill])
    xp = jnp.pad(x, ((0, PAD_N - N), (0, 0)))

    zh = jnp.zeros((PAD_N, H), jnp.float32)

    deg2 = _deg_call(dst.reshape(NW, EPW))
    g1, dis = _dense1(deg2.reshape(NW, PAD_N), xp, W1)
    src2 = src.reshape(NW, EPW)
    bank_off = (jnp.arange(E_PAD, dtype=jnp.int32) // CH % NB) * PAD_N
    dst2 = (dst + bank_off).reshape(NW, EPW)
    s1 = _scatter_h(src2, dst2, g1, zh)
    g2 = _dense2(s1.reshape(NC * NB, PAD_N, H), g1, dis, b1.reshape(1, H))
    s2 = _scatter_h(src2, dst2, g2, zh)
    s2 = s2.reshape(NC * NB, PAD_N, H)
    out = _dense3(s2, g2, dis, W2, b2.reshape(1, C))
    return out[:N]
